# baseline (speedup 1.0000x reference)
import jax
import jax.numpy as jnp
from jax import lax
from jax.experimental import pallas as pl
from jax.experimental.pallas import tpu as pltpu
from jax.experimental.pallas import tpu_sc as plsc

B = 16384
V = 1000000
R = 1000
D = 64

NC = 2
NS = 16
L = 16
NW = NC * NS
BPW = B // NW
CHUNK = 128
NCHUNK = BPW // CHUNK
GPC = CHUNK // L


def _rsqrt(x):
    i = plsc.bitcast(x, jnp.int32)
    y = plsc.bitcast(jnp.int32(0x5F3759DF) - (i >> 1), jnp.float32)
    xh = x * 0.5
    for _ in range(3):
        y = y * (1.5 - (xh * y) * y)
    return y


def _trans_e_body(hi2_hbm, hp_hbm, li2_hbm, lp_hbm, ti2_hbm, tp_hbm,
                  ent_hbm, rel_hbm, out_hbm,
                  idx_h, par_h, idx_l, par_l, idx_t, par_t,
                  hbuf, tbuf, rbuf, outv, sem):
    wid = lax.axis_index("s") * NC + lax.axis_index("c")

    pltpu.sync_copy(hi2_hbm.at[wid], idx_h)
    pltpu.sync_copy(hp_hbm.at[wid], par_h)
    pltpu.sync_copy(li2_hbm.at[wid], idx_l)
    pltpu.sync_copy(lp_hbm.at[wid], par_l)
    pltpu.sync_copy(ti2_hbm.at[wid], idx_t)
    pltpu.sync_copy(tp_hbm.at[wid], par_t)

    for c in range(NCHUNK):
        descs = [
            pltpu.async_copy(ent_hbm.at[idx_h.at[c]], hbuf, sem),
            pltpu.async_copy(ent_hbm.at[idx_t.at[c]], tbuf, sem),
            pltpu.async_copy(rel_hbm.at[idx_l.at[c]], rbuf, sem),
        ]
        for d in descs:
            d.wait()

        def group(g, carry, c=c):
            row = g * L + lax.iota(jnp.int32, L)
            ph = par_h[c, pl.ds(g * L, L)]
            pt = par_t[c, pl.ds(g * L, L)]
            prl = par_l[c, pl.ds(g * L, L)]
            zero = jnp.zeros((L,), jnp.float32)
            hh = zero; tt = zero; rr = zero
            hr = zero; ht = zero; rt = zero
            for j in range(D):
                h = plsc.load_gather(hbuf, [row, ph + j])
                t = plsc.load_gather(tbuf, [row, pt + j])
                r = plsc.load_gather(rbuf, [row, prl + j])
                hh = hh + h * h
                tt = tt + t * t
                rr = rr + r * r
                hr = hr + h * r
                ht = ht + h * t
                rt = rt + t * r
            a = 1.0 / jnp.maximum(hh * _rsqrt(hh), 1e-12)
            b = 1.0 / jnp.maximum(tt * _rsqrt(tt), 1e-12)
            d2 = ((a * hh) * a + rr + (b * tt) * b
                  + 2.0 * (a * hr) - 2.0 * ((a * ht) * b) - 2.0 * (b * rt))
            d2 = jnp.maximum(d2, 0.0)
            plsc.store_scatter(outv, [c * CHUNK + row], d2 * _rsqrt(d2))
            return carry

        lax.fori_loop(0, GPC, group, 0)

    pltpu.sync_copy(outv, out_hbm.at[pl.ds(wid * BPW, BPW)])


@jax.jit
def kernel(head_ind, label, tail_ind, ent_embs, rel_embs):
    mesh = plsc.VectorSubcoreMesh(core_axis_name="c", subcore_axis_name="s")
    run = pl.kernel(
        _trans_e_body,
        mesh=mesh,
        compiler_params=pltpu.CompilerParams(needs_layout_passes=False),
        out_type=jax.ShapeDtypeStruct((B,), jnp.float32),
        scratch_types=[
            pltpu.VMEM((NCHUNK, CHUNK), jnp.int32),
            pltpu.VMEM((NCHUNK, CHUNK), jnp.int32),
            pltpu.VMEM((NCHUNK, CHUNK), jnp.int32),
            pltpu.VMEM((NCHUNK, CHUNK), jnp.int32),
            pltpu.VMEM((NCHUNK, CHUNK), jnp.int32),
            pltpu.VMEM((NCHUNK, CHUNK), jnp.int32),
            pltpu.VMEM((CHUNK, 2 * D), jnp.float32),
            pltpu.VMEM((CHUNK, 2 * D), jnp.float32),
            pltpu.VMEM((CHUNK, 2 * D), jnp.float32),
            pltpu.VMEM((BPW,), jnp.float32),
            pltpu.SemaphoreType.DMA,
        ],
    )

    def prep(ind):
        i = ind.astype(jnp.int32)
        return ((i >> 1).reshape(NW, NCHUNK, CHUNK),
                ((i & 1) << 6).reshape(NW, NCHUNK, CHUNK))

    hi2, hp = prep(head_ind)
    li2, lp = prep(label)
    ti2, tp = prep(tail_ind)
    e2 = ent_embs.reshape(V // 2, 2 * D)
    r2 = rel_embs.reshape(R // 2, 2 * D)
    return run(hi2, hp, li2, lp, ti2, tp, e2, r2)
